# baseline (speedup 1.0000x reference)
import jax
import jax.numpy as jnp
from jax.experimental import pallas as pl


def _fused_t_kernel(at_ref, bt_ref, x_ref, ao_ref, bo_ref):
    a = at_ref[...]
    b = bt_ref[...]
    ao_ref[...] = a
    bo_ref[...] = b
    x_ref[...] = jnp.sum(a * b, axis=0)


def kernel(gum, gim):
    n, d = gum.shape
    at = gum.T
    bt = gim.T
    blk = 1024
    grid = n // blk
    x, aot, bot = pl.pallas_call(
        _fused_t_kernel,
        grid=(grid,),
        in_specs=[
            pl.BlockSpec((d, blk), lambda i: (0, i)),
            pl.BlockSpec((d, blk), lambda i: (0, i)),
        ],
        out_specs=(
            pl.BlockSpec((blk,), lambda i: (i,)),
            pl.BlockSpec((d, blk), lambda i: (0, i)),
            pl.BlockSpec((d, blk), lambda i: (0, i)),
        ),
        out_shape=(
            jax.ShapeDtypeStruct((n,), jnp.float32),
            jax.ShapeDtypeStruct((d, n), jnp.float32),
            jax.ShapeDtypeStruct((d, n), jnp.float32),
        ),
    )(at, bt)
    return (x, aot.T, bot.T)


# speedup vs baseline: 1.3217x; 1.3217x over previous
import jax
import jax.numpy as jnp
from jax.experimental import pallas as pl


def _fused_t_kernel(at_ref, bt_ref, x_ref, ao_ref, bo_ref):
    a = at_ref[...]
    b = bt_ref[...]
    ao_ref[...] = a
    bo_ref[...] = b
    x_ref[...] = jnp.sum(a * b, axis=0)


def kernel(gum, gim):
    n, d = gum.shape
    at = gum.T
    bt = gim.T
    blk = 2048
    grid = n // blk
    x, aot, bot = pl.pallas_call(
        _fused_t_kernel,
        grid=(grid,),
        in_specs=[
            pl.BlockSpec((d, blk), lambda i: (0, i)),
            pl.BlockSpec((d, blk), lambda i: (0, i)),
        ],
        out_specs=(
            pl.BlockSpec((blk,), lambda i: (i,)),
            pl.BlockSpec((d, blk), lambda i: (0, i)),
            pl.BlockSpec((d, blk), lambda i: (0, i)),
        ),
        out_shape=(
            jax.ShapeDtypeStruct((n,), jnp.float32),
            jax.ShapeDtypeStruct((d, n), jnp.float32),
            jax.ShapeDtypeStruct((d, n), jnp.float32),
        ),
    )(at, bt)
    return (x, aot.T, bot.T)
